# Initial kernel scaffold; baseline (speedup 1.0000x reference)
import jax
import jax.numpy as jnp
from jax.experimental import pallas as pl


def kernel(x, params):
    raise NotImplementedError("write your pallas kernel here")


# speedup vs baseline: 1.3993x; 1.3993x over previous
import functools

import jax
import jax.numpy as jnp
from jax import lax
from jax.experimental import pallas as pl

NUM_NODES = 800
BATCH = 2
SEQ = 8
HID = 8
OUT = 16
HEADS = 4

_HI = lax.Precision.HIGHEST


def _leaky_relu(x):
    return jnp.where(x >= 0, x, 0.2 * x)


def _gelu(x):
    return 0.5 * x * (1.0 + lax.erf(x / jnp.sqrt(2.0).astype(x.dtype)))


def _gat_block(hp, d_col, s_row, mask, head_dim):
    outs = []
    for h in range(HEADS):
        e = d_col[:, h:h + 1] + s_row[h:h + 1, :]
        e = _leaky_relu(e)
        e = jnp.where(mask, e, -1e30)
        m = jnp.max(e, axis=1, keepdims=True)
        w = jnp.where(mask, jnp.exp(e - m), 0.0)
        den = jnp.sum(w, axis=1, keepdims=True)
        alpha = w / den
        outs.append(jnp.dot(alpha, hp[:, h * head_dim:(h + 1) * head_dim],
                            preferred_element_type=jnp.float32))
    return jnp.concatenate(outs, axis=1)


def _stage_a_body(x_ref, emb_ref, prjW_ref, prjb_ref,
                  W1_ref, As1_ref, Ad1_ref, b1_ref,
                  W2_ref, As2_ref, Ad2_ref, b2_ref, out_ref):
    xb = x_ref[0]
    emb = emb_ref[...]
    nrm = jnp.sqrt(jnp.sum(emb * emb, axis=1, keepdims=True)) + 1e-12
    ne = emb / nrm
    adj = lax.dot_general(ne, ne, (((1,), (1,)), ((), ())),
                          precision=_HI, preferred_element_type=jnp.float32)
    mask = adj > 0.5

    h0 = jnp.dot(xb, prjW_ref[...], precision=_HI) + prjb_ref[...]

    hp1 = jnp.dot(h0, W1_ref[...], precision=_HI)
    d1 = jnp.dot(hp1, Ad1_ref[...], precision=_HI)
    s1 = lax.dot_general(As1_ref[...], hp1, (((0,), (1,)), ((), ())),
                         precision=_HI)
    g1 = _gat_block(hp1, d1, s1, mask, HID // HEADS) + b1_ref[...]
    g1 = _gelu(g1)

    hp2 = jnp.dot(g1, W2_ref[...], precision=_HI)
    d2 = jnp.dot(hp2, Ad2_ref[...], precision=_HI)
    s2 = lax.dot_general(As2_ref[...], hp2, (((0,), (1,)), ((), ())),
                         precision=_HI)
    g2 = _gat_block(hp2, d2, s2, mask, OUT // HEADS) + b2_ref[...]

    out_ref[0] = g2


def _stage_b_body(t_ref, wq_ref, bq_ref, wk_ref, bk_ref, wv_ref, bv_ref,
                  wo_ref, bo_ref, w1_ref, b1_ref, w2_ref, b2_ref,
                  ln1g_ref, ln1b_ref, ln2g_ref, ln2b_ref,
                  rw1_ref, rb1_ref, rw2_ref, rb2_ref, out_ref):
    T = t_ref[...]
    Q = jnp.dot(wq_ref[...], T, precision=_HI) + bq_ref[...]
    K = jnp.dot(wk_ref[...], T, precision=_HI) + bk_ref[...]
    V = jnp.dot(wv_ref[...], T, precision=_HI) + bv_ref[...]
    inv = 1.0 / jnp.sqrt(8.0)

    ao_rows = []
    for qi in range(SEQ):
        qs = Q[qi * 16:(qi + 1) * 16]
        sc = []
        for ki in range(SEQ):
            prod = qs * K[ki * 16:(ki + 1) * 16]
            sc.append([jnp.sum(prod[h * 8:(h + 1) * 8], axis=0, keepdims=True)
                       * inv for h in range(2)])
        ao_h = []
        for h in range(2):
            m = sc[0][h]
            for ki in range(1, SEQ):
                m = jnp.maximum(m, sc[ki][h])
            exps = [jnp.exp(sc[ki][h] - m) for ki in range(SEQ)]
            den = exps[0]
            for ki in range(1, SEQ):
                den = den + exps[ki]
            acc = jnp.zeros((8, T.shape[1]), jnp.float32)
            for ki in range(SEQ):
                a = exps[ki] / den
                acc = acc + a * V[ki * 16 + h * 8: ki * 16 + h * 8 + 8]
            ao_h.append(acc)
        ao_rows.append(jnp.concatenate(ao_h, axis=0))
    AO = jnp.concatenate(ao_rows, axis=0)

    X = T + jnp.dot(wo_ref[...], AO, precision=_HI) + bo_ref[...]

    def _ln(x, g, b):
        rows = []
        for s in range(SEQ):
            blk = x[s * 16:(s + 1) * 16]
            m = jnp.mean(blk, axis=0, keepdims=True)
            c = blk - m
            v = jnp.mean(c * c, axis=0, keepdims=True)
            rows.append(c / jnp.sqrt(v + 1e-3))
        return jnp.concatenate(rows, axis=0) * g + b

    T1 = _ln(X, ln1g_ref[...], ln1b_ref[...])
    F = jnp.dot(w2_ref[...],
                _gelu(jnp.dot(w1_ref[...], T1, precision=_HI) + b1_ref[...]),
                precision=_HI) + b2_ref[...]
    T2 = _ln(T1 + F, ln2g_ref[...], ln2b_ref[...])

    P = T2[0:16]
    for s in range(1, SEQ):
        P = P + T2[s * 16:(s + 1) * 16]
    P = P / float(SEQ)

    R = _gelu(jnp.dot(rw1_ref[...], P, precision=_HI) + rb1_ref[...])
    out = jnp.sum(R * rw2_ref[...], axis=0, keepdims=True) + rb2_ref[...]
    out_ref[...] = out


def _block_diag_cols(a):
    H, D = a.shape
    out = jnp.zeros((H * D, H), jnp.float32)
    for h in range(H):
        out = out.at[h * D:(h + 1) * D, h].set(a[h])
    return out


def kernel(x, params):
    p = params
    B, S, N, F = x.shape
    xs = x.reshape(B * S, N, F)
    xp = jnp.concatenate(
        [xs, jnp.zeros((B * S, N, HID - F), jnp.float32)], axis=-1)
    prjW = jnp.concatenate(
        [p['proj_W'], jnp.zeros((HID - F, HID), jnp.float32)], axis=0)

    W1 = p['gat1_W'].reshape(HID, HID)
    As1 = _block_diag_cols(p['gat1_as'])
    Ad1 = _block_diag_cols(p['gat1_ad'])
    W2 = p['gat2_W'].reshape(HID, OUT)
    As2 = _block_diag_cols(p['gat2_as'])
    Ad2 = _block_diag_cols(p['gat2_ad'])

    full = lambda shape: pl.BlockSpec(shape, lambda g: tuple(0 for _ in shape))
    h_out = pl.pallas_call(
        _stage_a_body,
        grid=(B * S,),
        in_specs=[
            pl.BlockSpec((1, N, HID), lambda g: (g, 0, 0)),
            full((N, HID)), full((HID, HID)), full((1, HID)),
            full((HID, HID)), full((HID, HEADS)), full((HID, HEADS)),
            full((1, HID)),
            full((HID, OUT)), full((OUT, HEADS)), full((OUT, HEADS)),
            full((1, OUT)),
        ],
        out_specs=pl.BlockSpec((1, N, OUT), lambda g: (g, 0, 0)),
        out_shape=jax.ShapeDtypeStruct((B * S, N, OUT), jnp.float32),
    )(xp, p['emb'], prjW, p['proj_b'].reshape(1, HID),
      W1, As1, Ad1, p['gat1_b'].reshape(1, HID),
      W2, As2, Ad2, p['gat2_b'].reshape(1, OUT))

    L = B * N
    Tin = h_out.reshape(L, SEQ, OUT).transpose(1, 2, 0).reshape(SEQ * OUT, L)

    eye8 = jnp.eye(SEQ, dtype=jnp.float32)
    bd = lambda w: jnp.kron(eye8, w.T)
    col = lambda v, rep: jnp.tile(v.reshape(-1), rep).reshape(-1, 1)

    Wq = bd(p['Wq'].reshape(16, 16))
    Wk = bd(p['Wk'].reshape(16, 16))
    Wv = bd(p['Wv'].reshape(16, 16))
    Wo = bd(p['Wo'].reshape(16, 16))
    W1f = bd(p['ffn_W1'])
    W2f = bd(p['ffn_W2'])

    out = pl.pallas_call(
        _stage_b_body,
        in_specs=[
            full((SEQ * OUT, L)),
            full((128, 128)), full((128, 1)),
            full((128, 128)), full((128, 1)),
            full((128, 128)), full((128, 1)),
            full((128, 128)), full((128, 1)),
            full((256, 128)), full((256, 1)),
            full((128, 256)), full((128, 1)),
            full((128, 1)), full((128, 1)), full((128, 1)), full((128, 1)),
            full((16, 16)), full((16, 1)), full((16, 1)), full((1, 1)),
        ],
        out_specs=full((1, L)),
        out_shape=jax.ShapeDtypeStruct((1, L), jnp.float32),
        grid=(1,),
    )(Tin,
      Wq, col(p['bq'], SEQ), Wk, col(p['bk'], SEQ), Wv, col(p['bv'], SEQ),
      Wo, col(p['bo'], SEQ),
      W1f, col(p['ffn_b1'], SEQ), W2f, col(p['ffn_b2'], SEQ),
      col(p['ln1_g'], SEQ), col(p['ln1_b'], SEQ),
      col(p['ln2_g'], SEQ), col(p['ln2_b'], SEQ),
      p['reg_W1'].T, p['reg_b1'].reshape(16, 1),
      p['reg_W2'].reshape(16, 1), p['reg_b2'].reshape(1, 1))

    return out.reshape(B, N)


# speedup vs baseline: 2.0713x; 1.4803x over previous
import functools

import jax
import jax.numpy as jnp
from jax import lax
from jax.experimental import pallas as pl
from jax.experimental.pallas import tpu as pltpu

NUM_NODES = 800
BATCH = 2
SEQ = 8
HID = 8
OUT = 16
HEADS = 4

_HI = lax.Precision.HIGHEST


def _leaky_relu(x):
    return jnp.maximum(x, 0.2 * x)


def _gelu(x):
    return 0.5 * x * (1.0 + lax.erf(x / jnp.sqrt(2.0).astype(x.dtype)))


def _gat_block(hp, d_col, s_row, neg, ones_col, head_dim):
    outs = []
    dens = []
    for h in range(HEADS):
        e = _leaky_relu(d_col[:, h:h + 1] + s_row[h:h + 1, :]) + neg
        m = jnp.max(e, axis=1, keepdims=True)
        w = jnp.exp(e - m)
        aug = jnp.concatenate(
            [hp[:, h * head_dim:(h + 1) * head_dim], ones_col], axis=1)
        od = jnp.dot(w, aug, preferred_element_type=jnp.float32)
        outs.append(od[:, :head_dim])
        dens.append(od[:, head_dim:head_dim + 1])
    den = jnp.concatenate(
        [jnp.broadcast_to(d, (d.shape[0], head_dim)) for d in dens], axis=1)
    return jnp.concatenate(outs, axis=1) / den


def _stage_a_body(x_ref, emb_ref, prjW_ref, prjb_ref,
                  W1_ref, As1_ref, Ad1_ref, b1_ref,
                  W2_ref, As2_ref, Ad2_ref, b2_ref, out_ref, neg_ref):
    @pl.when(pl.program_id(0) == 0)
    def _():
        emb = emb_ref[...]
        nrm = jnp.sqrt(jnp.sum(emb * emb, axis=1, keepdims=True)) + 1e-12
        ne = emb / nrm
        adj = lax.dot_general(ne, ne, (((1,), (1,)), ((), ())),
                              precision=_HI, preferred_element_type=jnp.float32)
        neg_ref[...] = jnp.where(adj > 0.5, 0.0, -1e30)

    xb = x_ref[0]
    neg = neg_ref[...]
    ones_col = jnp.ones((NUM_NODES, 1), jnp.float32)

    h0 = jnp.dot(xb, prjW_ref[...], precision=_HI) + prjb_ref[...]

    hp1 = jnp.dot(h0, W1_ref[...], precision=_HI)
    d1 = jnp.dot(hp1, Ad1_ref[...], precision=_HI)
    s1 = lax.dot_general(As1_ref[...], hp1, (((0,), (1,)), ((), ())),
                         precision=_HI)
    g1 = _gat_block(hp1, d1, s1, neg, ones_col, HID // HEADS) + b1_ref[...]
    g1 = _gelu(g1)

    hp2 = jnp.dot(g1, W2_ref[...], precision=_HI)
    d2 = jnp.dot(hp2, Ad2_ref[...], precision=_HI)
    s2 = lax.dot_general(As2_ref[...], hp2, (((0,), (1,)), ((), ())),
                         precision=_HI)
    g2 = _gat_block(hp2, d2, s2, neg, ones_col, OUT // HEADS) + b2_ref[...]

    out_ref[0] = g2


def _stage_b_body(t_ref, wq_ref, bq_ref, wk_ref, bk_ref, wv_ref, bv_ref,
                  wo_ref, bo_ref, w1_ref, b1_ref, w2_ref, b2_ref,
                  ln1g_ref, ln1b_ref, ln2g_ref, ln2b_ref,
                  rw1_ref, rb1_ref, rw2_ref, rb2_ref, out_ref):
    T = t_ref[...]
    Q = jnp.dot(wq_ref[...], T, precision=_HI) + bq_ref[...]
    K = jnp.dot(wk_ref[...], T, precision=_HI) + bk_ref[...]
    V = jnp.dot(wv_ref[...], T, precision=_HI) + bv_ref[...]
    inv = 1.0 / jnp.sqrt(8.0)

    ao_rows = []
    for qi in range(SEQ):
        qs = Q[qi * 16:(qi + 1) * 16]
        sc = []
        for ki in range(SEQ):
            prod = qs * K[ki * 16:(ki + 1) * 16]
            sc.append([jnp.sum(prod[h * 8:(h + 1) * 8], axis=0, keepdims=True)
                       * inv for h in range(2)])
        ao_h = []
        for h in range(2):
            m = sc[0][h]
            for ki in range(1, SEQ):
                m = jnp.maximum(m, sc[ki][h])
            exps = [jnp.exp(sc[ki][h] - m) for ki in range(SEQ)]
            den = exps[0]
            for ki in range(1, SEQ):
                den = den + exps[ki]
            acc = jnp.zeros((8, T.shape[1]), jnp.float32)
            for ki in range(SEQ):
                a = exps[ki] / den
                acc = acc + a * V[ki * 16 + h * 8: ki * 16 + h * 8 + 8]
            ao_h.append(acc)
        ao_rows.append(jnp.concatenate(ao_h, axis=0))
    AO = jnp.concatenate(ao_rows, axis=0)

    X = T + jnp.dot(wo_ref[...], AO, precision=_HI) + bo_ref[...]

    def _ln(x, g, b):
        rows = []
        for s in range(SEQ):
            blk = x[s * 16:(s + 1) * 16]
            m = jnp.mean(blk, axis=0, keepdims=True)
            c = blk - m
            v = jnp.mean(c * c, axis=0, keepdims=True)
            rows.append(c / jnp.sqrt(v + 1e-3))
        return jnp.concatenate(rows, axis=0) * g + b

    T1 = _ln(X, ln1g_ref[...], ln1b_ref[...])
    F = jnp.dot(w2_ref[...],
                _gelu(jnp.dot(w1_ref[...], T1, precision=_HI) + b1_ref[...]),
                precision=_HI) + b2_ref[...]
    T2 = _ln(T1 + F, ln2g_ref[...], ln2b_ref[...])

    P = T2[0:16]
    for s in range(1, SEQ):
        P = P + T2[s * 16:(s + 1) * 16]
    P = P / float(SEQ)

    R = _gelu(jnp.dot(rw1_ref[...], P, precision=_HI) + rb1_ref[...])
    out = jnp.sum(R * rw2_ref[...], axis=0, keepdims=True) + rb2_ref[...]
    out_ref[...] = out


def _block_diag_cols(a):
    H, D = a.shape
    out = jnp.zeros((H * D, H), jnp.float32)
    for h in range(H):
        out = out.at[h * D:(h + 1) * D, h].set(a[h])
    return out


def kernel(x, params):
    p = params
    B, S, N, F = x.shape
    xs = x.reshape(B * S, N, F)
    xp = jnp.concatenate(
        [xs, jnp.zeros((B * S, N, HID - F), jnp.float32)], axis=-1)
    prjW = jnp.concatenate(
        [p['proj_W'], jnp.zeros((HID - F, HID), jnp.float32)], axis=0)

    W1 = p['gat1_W'].reshape(HID, HID)
    As1 = _block_diag_cols(p['gat1_as'])
    Ad1 = _block_diag_cols(p['gat1_ad'])
    W2 = p['gat2_W'].reshape(HID, OUT)
    As2 = _block_diag_cols(p['gat2_as'])
    Ad2 = _block_diag_cols(p['gat2_ad'])

    full = lambda shape: pl.BlockSpec(shape, lambda g: tuple(0 for _ in shape))
    h_out = pl.pallas_call(
        _stage_a_body,
        grid=(B * S,),
        in_specs=[
            pl.BlockSpec((1, N, HID), lambda g: (g, 0, 0)),
            full((N, HID)), full((HID, HID)), full((1, HID)),
            full((HID, HID)), full((HID, HEADS)), full((HID, HEADS)),
            full((1, HID)),
            full((HID, OUT)), full((OUT, HEADS)), full((OUT, HEADS)),
            full((1, OUT)),
        ],
        out_specs=pl.BlockSpec((1, N, OUT), lambda g: (g, 0, 0)),
        out_shape=jax.ShapeDtypeStruct((B * S, N, OUT), jnp.float32),
        scratch_shapes=[pltpu.VMEM((N, N), jnp.float32)],
    )(xp, p['emb'], prjW, p['proj_b'].reshape(1, HID),
      W1, As1, Ad1, p['gat1_b'].reshape(1, HID),
      W2, As2, Ad2, p['gat2_b'].reshape(1, OUT))

    L = B * N
    Tin = h_out.reshape(L, SEQ, OUT).transpose(1, 2, 0).reshape(SEQ * OUT, L)

    eye8 = jnp.eye(SEQ, dtype=jnp.float32)
    bd = lambda w: jnp.kron(eye8, w.T)
    col = lambda v, rep: jnp.tile(v.reshape(-1), rep).reshape(-1, 1)

    Wq = bd(p['Wq'].reshape(16, 16))
    Wk = bd(p['Wk'].reshape(16, 16))
    Wv = bd(p['Wv'].reshape(16, 16))
    Wo = bd(p['Wo'].reshape(16, 16))
    W1f = bd(p['ffn_W1'])
    W2f = bd(p['ffn_W2'])

    out = pl.pallas_call(
        _stage_b_body,
        in_specs=[
            full((SEQ * OUT, L)),
            full((128, 128)), full((128, 1)),
            full((128, 128)), full((128, 1)),
            full((128, 128)), full((128, 1)),
            full((128, 128)), full((128, 1)),
            full((256, 128)), full((256, 1)),
            full((128, 256)), full((128, 1)),
            full((128, 1)), full((128, 1)), full((128, 1)), full((128, 1)),
            full((16, 16)), full((16, 1)), full((16, 1)), full((1, 1)),
        ],
        out_specs=full((1, L)),
        out_shape=jax.ShapeDtypeStruct((1, L), jnp.float32),
        grid=(1,),
    )(Tin,
      Wq, col(p['bq'], SEQ), Wk, col(p['bk'], SEQ), Wv, col(p['bv'], SEQ),
      Wo, col(p['bo'], SEQ),
      W1f, col(p['ffn_b1'], SEQ), W2f, col(p['ffn_b2'], SEQ),
      col(p['ln1_g'], SEQ), col(p['ln1_b'], SEQ),
      col(p['ln2_g'], SEQ), col(p['ln2_b'], SEQ),
      p['reg_W1'].T, p['reg_b1'].reshape(16, 1),
      p['reg_W2'].reshape(16, 1), p['reg_b2'].reshape(1, 1))

    return out.reshape(B, N)


# speedup vs baseline: 2.0792x; 1.0038x over previous
import functools

import jax
import jax.numpy as jnp
from jax import lax
from jax.experimental import pallas as pl
from jax.experimental.pallas import tpu as pltpu

NUM_NODES = 800
BATCH = 2
SEQ = 8
HID = 8
OUT = 16
HEADS = 4

_HI = lax.Precision.HIGHEST


def _leaky_relu(x):
    return jnp.maximum(x, 0.2 * x)


def _gelu(x):
    return 0.5 * x * (1.0 + lax.erf(x / jnp.sqrt(2.0).astype(x.dtype)))


def _gat_block(hp, d_col, s_row, neg, ones_col, head_dim):
    outs = []
    dens = []
    for h in range(HEADS):
        e = _leaky_relu(d_col[:, h:h + 1] + s_row[h:h + 1, :]) + neg
        m = jnp.max(e, axis=1, keepdims=True)
        w = jnp.exp(e - m)
        aug = jnp.concatenate(
            [hp[:, h * head_dim:(h + 1) * head_dim], ones_col], axis=1)
        od = jnp.dot(w.astype(jnp.bfloat16), aug.astype(jnp.bfloat16),
                     preferred_element_type=jnp.float32)
        outs.append(od[:, :head_dim])
        dens.append(od[:, head_dim:head_dim + 1])
    den = jnp.concatenate(
        [jnp.broadcast_to(d, (d.shape[0], head_dim)) for d in dens], axis=1)
    return jnp.concatenate(outs, axis=1) / den


def _stage_a_body(x_ref, emb_ref, prjW_ref, prjb_ref,
                  W1_ref, As1_ref, Ad1_ref, b1_ref,
                  W2_ref, As2_ref, Ad2_ref, b2_ref, out_ref, neg_ref):
    @pl.when(pl.program_id(0) == 0)
    def _():
        emb = emb_ref[...]
        nrm = jnp.sqrt(jnp.sum(emb * emb, axis=1, keepdims=True)) + 1e-12
        ne = emb / nrm
        adj = lax.dot_general(ne, ne, (((1,), (1,)), ((), ())),
                              precision=_HI, preferred_element_type=jnp.float32)
        neg_ref[...] = jnp.where(adj > 0.5, 0.0, -1e30)

    xb = x_ref[0]
    neg = neg_ref[...]
    ones_col = jnp.ones((NUM_NODES, 1), jnp.float32)

    h0 = jnp.dot(xb, prjW_ref[...], precision=_HI) + prjb_ref[...]

    hp1 = jnp.dot(h0, W1_ref[...], precision=_HI)
    d1 = jnp.dot(hp1, Ad1_ref[...], precision=_HI)
    s1 = lax.dot_general(As1_ref[...], hp1, (((0,), (1,)), ((), ())),
                         precision=_HI)
    g1 = _gat_block(hp1, d1, s1, neg, ones_col, HID // HEADS) + b1_ref[...]
    g1 = _gelu(g1)

    hp2 = jnp.dot(g1, W2_ref[...], precision=_HI)
    d2 = jnp.dot(hp2, Ad2_ref[...], precision=_HI)
    s2 = lax.dot_general(As2_ref[...], hp2, (((0,), (1,)), ((), ())),
                         precision=_HI)
    g2 = _gat_block(hp2, d2, s2, neg, ones_col, OUT // HEADS) + b2_ref[...]

    out_ref[0] = g2


def _stage_b_body(t_ref, wq_ref, bq_ref, wk_ref, bk_ref, wv_ref, bv_ref,
                  wo_ref, bo_ref, w1_ref, b1_ref, w2_ref, b2_ref,
                  ln1g_ref, ln1b_ref, ln2g_ref, ln2b_ref,
                  rw1_ref, rb1_ref, rw2_ref, rb2_ref, out_ref):
    T = t_ref[...]
    Q = jnp.dot(wq_ref[...], T, precision=_HI) + bq_ref[...]
    K = jnp.dot(wk_ref[...], T, precision=_HI) + bk_ref[...]
    V = jnp.dot(wv_ref[...], T, precision=_HI) + bv_ref[...]
    inv = 1.0 / jnp.sqrt(8.0)

    ao_rows = []
    for qi in range(SEQ):
        qs = Q[qi * 16:(qi + 1) * 16]
        sc = []
        for ki in range(SEQ):
            prod = qs * K[ki * 16:(ki + 1) * 16]
            sc.append([jnp.sum(prod[h * 8:(h + 1) * 8], axis=0, keepdims=True)
                       * inv for h in range(2)])
        ao_h = []
        for h in range(2):
            m = sc[0][h]
            for ki in range(1, SEQ):
                m = jnp.maximum(m, sc[ki][h])
            exps = [jnp.exp(sc[ki][h] - m) for ki in range(SEQ)]
            den = exps[0]
            for ki in range(1, SEQ):
                den = den + exps[ki]
            acc = jnp.zeros((8, T.shape[1]), jnp.float32)
            for ki in range(SEQ):
                a = exps[ki] / den
                acc = acc + a * V[ki * 16 + h * 8: ki * 16 + h * 8 + 8]
            ao_h.append(acc)
        ao_rows.append(jnp.concatenate(ao_h, axis=0))
    AO = jnp.concatenate(ao_rows, axis=0)

    X = T + jnp.dot(wo_ref[...], AO, precision=_HI) + bo_ref[...]

    def _ln(x, g, b):
        rows = []
        for s in range(SEQ):
            blk = x[s * 16:(s + 1) * 16]
            m = jnp.mean(blk, axis=0, keepdims=True)
            c = blk - m
            v = jnp.mean(c * c, axis=0, keepdims=True)
            rows.append(c / jnp.sqrt(v + 1e-3))
        return jnp.concatenate(rows, axis=0) * g + b

    T1 = _ln(X, ln1g_ref[...], ln1b_ref[...])
    F = jnp.dot(w2_ref[...],
                _gelu(jnp.dot(w1_ref[...], T1, precision=_HI) + b1_ref[...]),
                precision=_HI) + b2_ref[...]
    T2 = _ln(T1 + F, ln2g_ref[...], ln2b_ref[...])

    P = T2[0:16]
    for s in range(1, SEQ):
        P = P + T2[s * 16:(s + 1) * 16]
    P = P / float(SEQ)

    R = _gelu(jnp.dot(rw1_ref[...], P, precision=_HI) + rb1_ref[...])
    out = jnp.sum(R * rw2_ref[...], axis=0, keepdims=True) + rb2_ref[...]
    out_ref[...] = out


def _block_diag_cols(a):
    H, D = a.shape
    out = jnp.zeros((H * D, H), jnp.float32)
    for h in range(H):
        out = out.at[h * D:(h + 1) * D, h].set(a[h])
    return out


def kernel(x, params):
    p = params
    B, S, N, F = x.shape
    xs = x.reshape(B * S, N, F)
    xp = jnp.concatenate(
        [xs, jnp.zeros((B * S, N, HID - F), jnp.float32)], axis=-1)
    prjW = jnp.concatenate(
        [p['proj_W'], jnp.zeros((HID - F, HID), jnp.float32)], axis=0)

    W1 = p['gat1_W'].reshape(HID, HID)
    As1 = _block_diag_cols(p['gat1_as'])
    Ad1 = _block_diag_cols(p['gat1_ad'])
    W2 = p['gat2_W'].reshape(HID, OUT)
    As2 = _block_diag_cols(p['gat2_as'])
    Ad2 = _block_diag_cols(p['gat2_ad'])

    full = lambda shape: pl.BlockSpec(shape, lambda g: tuple(0 for _ in shape))
    h_out = pl.pallas_call(
        _stage_a_body,
        grid=(B * S,),
        in_specs=[
            pl.BlockSpec((1, N, HID), lambda g: (g, 0, 0)),
            full((N, HID)), full((HID, HID)), full((1, HID)),
            full((HID, HID)), full((HID, HEADS)), full((HID, HEADS)),
            full((1, HID)),
            full((HID, OUT)), full((OUT, HEADS)), full((OUT, HEADS)),
            full((1, OUT)),
        ],
        out_specs=pl.BlockSpec((1, N, OUT), lambda g: (g, 0, 0)),
        out_shape=jax.ShapeDtypeStruct((B * S, N, OUT), jnp.float32),
        scratch_shapes=[pltpu.VMEM((N, N), jnp.float32)],
    )(xp, p['emb'], prjW, p['proj_b'].reshape(1, HID),
      W1, As1, Ad1, p['gat1_b'].reshape(1, HID),
      W2, As2, Ad2, p['gat2_b'].reshape(1, OUT))

    L = B * N
    Tin = h_out.reshape(L, SEQ, OUT).transpose(1, 2, 0).reshape(SEQ * OUT, L)

    eye8 = jnp.eye(SEQ, dtype=jnp.float32)
    bd = lambda w: jnp.kron(eye8, w.T)
    col = lambda v, rep: jnp.tile(v.reshape(-1), rep).reshape(-1, 1)

    Wq = bd(p['Wq'].reshape(16, 16))
    Wk = bd(p['Wk'].reshape(16, 16))
    Wv = bd(p['Wv'].reshape(16, 16))
    Wo = bd(p['Wo'].reshape(16, 16))
    W1f = bd(p['ffn_W1'])
    W2f = bd(p['ffn_W2'])

    out = pl.pallas_call(
        _stage_b_body,
        in_specs=[
            full((SEQ * OUT, L)),
            full((128, 128)), full((128, 1)),
            full((128, 128)), full((128, 1)),
            full((128, 128)), full((128, 1)),
            full((128, 128)), full((128, 1)),
            full((256, 128)), full((256, 1)),
            full((128, 256)), full((128, 1)),
            full((128, 1)), full((128, 1)), full((128, 1)), full((128, 1)),
            full((16, 16)), full((16, 1)), full((16, 1)), full((1, 1)),
        ],
        out_specs=full((1, L)),
        out_shape=jax.ShapeDtypeStruct((1, L), jnp.float32),
        grid=(1,),
    )(Tin,
      Wq, col(p['bq'], SEQ), Wk, col(p['bk'], SEQ), Wv, col(p['bv'], SEQ),
      Wo, col(p['bo'], SEQ),
      W1f, col(p['ffn_b1'], SEQ), W2f, col(p['ffn_b2'], SEQ),
      col(p['ln1_g'], SEQ), col(p['ln1_b'], SEQ),
      col(p['ln2_g'], SEQ), col(p['ln2_b'], SEQ),
      p['reg_W1'].T, p['reg_b1'].reshape(16, 1),
      p['reg_W2'].reshape(16, 1), p['reg_b2'].reshape(1, 1))

    return out.reshape(B, N)


# speedup vs baseline: 2.3270x; 1.1192x over previous
import functools

import jax
import jax.numpy as jnp
from jax import lax
from jax.experimental import pallas as pl
from jax.experimental.pallas import tpu as pltpu

NUM_NODES = 800
BATCH = 2
SEQ = 8
HID = 8
OUT = 16
HEADS = 4

_HI = lax.Precision.HIGHEST


def _leaky_relu(x):
    return jnp.maximum(x, 0.2 * x)


def _gelu(x):
    return 0.5 * x * (1.0 + lax.erf(x / jnp.sqrt(2.0).astype(x.dtype)))


def _gat_block(hp, d_col, s_row, neg, ones_col, head_dim):
    outs = []
    dens = []
    log2e = jnp.float32(1.4426950408889634)
    dc = d_col * log2e
    sc = s_row * log2e
    for h in range(HEADS):
        t = dc[:, h:h + 1] + sc[h:h + 1, :]
        e = jnp.maximum(t, 0.2 * t) + neg
        w = jnp.exp2(jnp.minimum(e, 86.0))
        aug = jnp.concatenate(
            [hp[:, h * head_dim:(h + 1) * head_dim], ones_col], axis=1)
        od = jnp.dot(w, aug, preferred_element_type=jnp.float32)
        outs.append(od[:, :head_dim])
        dens.append(od[:, head_dim:head_dim + 1])
    den = jnp.concatenate(
        [jnp.broadcast_to(d, (d.shape[0], head_dim)) for d in dens], axis=1)
    return jnp.concatenate(outs, axis=1) / den


def _stage_a_body(x_ref, emb_ref, prjW_ref, prjb_ref,
                  W1_ref, As1_ref, Ad1_ref, b1_ref,
                  W2_ref, As2_ref, Ad2_ref, b2_ref, out_ref, neg_ref):
    @pl.when(pl.program_id(0) == 0)
    def _():
        emb = emb_ref[...]
        nrm = jnp.sqrt(jnp.sum(emb * emb, axis=1, keepdims=True)) + 1e-12
        ne = emb / nrm
        adj = lax.dot_general(ne, ne, (((1,), (1,)), ((), ())),
                              precision=_HI, preferred_element_type=jnp.float32)
        neg_ref[...] = jnp.where(adj > 0.5, 0.0, -1e30)

    xb = x_ref[0]
    neg = neg_ref[...]
    ones_col = jnp.ones((NUM_NODES, 1), jnp.float32)

    h0 = jnp.dot(xb, prjW_ref[...], precision=_HI) + prjb_ref[...]

    hp1 = jnp.dot(h0, W1_ref[...], precision=_HI)
    d1 = jnp.dot(hp1, Ad1_ref[...], precision=_HI)
    s1 = lax.dot_general(As1_ref[...], hp1, (((0,), (1,)), ((), ())),
                         precision=_HI)
    g1 = _gat_block(hp1, d1, s1, neg, ones_col, HID // HEADS) + b1_ref[...]
    g1 = _gelu(g1)

    hp2 = jnp.dot(g1, W2_ref[...], precision=_HI)
    d2 = jnp.dot(hp2, Ad2_ref[...], precision=_HI)
    s2 = lax.dot_general(As2_ref[...], hp2, (((0,), (1,)), ((), ())),
                         precision=_HI)
    g2 = _gat_block(hp2, d2, s2, neg, ones_col, OUT // HEADS) + b2_ref[...]

    out_ref[0] = g2


def _stage_b_body(t_ref, wq_ref, bq_ref, wk_ref, bk_ref, wv_ref, bv_ref,
                  wo_ref, bo_ref, w1_ref, b1_ref, w2_ref, b2_ref,
                  ln1g_ref, ln1b_ref, ln2g_ref, ln2b_ref,
                  rw1_ref, rb1_ref, rw2_ref, rb2_ref, out_ref):
    T = t_ref[...]
    Q = jnp.dot(wq_ref[...], T, precision=_HI) + bq_ref[...]
    K = jnp.dot(wk_ref[...], T, precision=_HI) + bk_ref[...]
    V = jnp.dot(wv_ref[...], T, precision=_HI) + bv_ref[...]
    inv = 1.0 / jnp.sqrt(8.0)

    ao_rows = []
    for qi in range(SEQ):
        qs = Q[qi * 16:(qi + 1) * 16]
        sc = []
        for ki in range(SEQ):
            prod = qs * K[ki * 16:(ki + 1) * 16]
            sc.append([jnp.sum(prod[h * 8:(h + 1) * 8], axis=0, keepdims=True)
                       * inv for h in range(2)])
        ao_h = []
        for h in range(2):
            m = sc[0][h]
            for ki in range(1, SEQ):
                m = jnp.maximum(m, sc[ki][h])
            exps = [jnp.exp(sc[ki][h] - m) for ki in range(SEQ)]
            den = exps[0]
            for ki in range(1, SEQ):
                den = den + exps[ki]
            acc = jnp.zeros((8, T.shape[1]), jnp.float32)
            for ki in range(SEQ):
                a = exps[ki] / den
                acc = acc + a * V[ki * 16 + h * 8: ki * 16 + h * 8 + 8]
            ao_h.append(acc)
        ao_rows.append(jnp.concatenate(ao_h, axis=0))
    AO = jnp.concatenate(ao_rows, axis=0)

    X = T + jnp.dot(wo_ref[...], AO, precision=_HI) + bo_ref[...]

    def _ln(x, g, b):
        rows = []
        for s in range(SEQ):
            blk = x[s * 16:(s + 1) * 16]
            m = jnp.mean(blk, axis=0, keepdims=True)
            c = blk - m
            v = jnp.mean(c * c, axis=0, keepdims=True)
            rows.append(c / jnp.sqrt(v + 1e-3))
        return jnp.concatenate(rows, axis=0) * g + b

    T1 = _ln(X, ln1g_ref[...], ln1b_ref[...])
    F = jnp.dot(w2_ref[...],
                _gelu(jnp.dot(w1_ref[...], T1, precision=_HI) + b1_ref[...]),
                precision=_HI) + b2_ref[...]
    T2 = _ln(T1 + F, ln2g_ref[...], ln2b_ref[...])

    P = T2[0:16]
    for s in range(1, SEQ):
        P = P + T2[s * 16:(s + 1) * 16]
    P = P / float(SEQ)

    R = _gelu(jnp.dot(rw1_ref[...], P, precision=_HI) + rb1_ref[...])
    out = jnp.sum(R * rw2_ref[...], axis=0, keepdims=True) + rb2_ref[...]
    out_ref[...] = out


def _block_diag_cols(a):
    H, D = a.shape
    out = jnp.zeros((H * D, H), jnp.float32)
    for h in range(H):
        out = out.at[h * D:(h + 1) * D, h].set(a[h])
    return out


def kernel(x, params):
    p = params
    B, S, N, F = x.shape
    xs = x.reshape(B * S, N, F)
    xp = jnp.concatenate(
        [xs, jnp.zeros((B * S, N, HID - F), jnp.float32)], axis=-1)
    prjW = jnp.concatenate(
        [p['proj_W'], jnp.zeros((HID - F, HID), jnp.float32)], axis=0)

    W1 = p['gat1_W'].reshape(HID, HID)
    As1 = _block_diag_cols(p['gat1_as'])
    Ad1 = _block_diag_cols(p['gat1_ad'])
    W2 = p['gat2_W'].reshape(HID, OUT)
    As2 = _block_diag_cols(p['gat2_as'])
    Ad2 = _block_diag_cols(p['gat2_ad'])

    full = lambda shape: pl.BlockSpec(shape, lambda g: tuple(0 for _ in shape))
    h_out = pl.pallas_call(
        _stage_a_body,
        grid=(B * S,),
        in_specs=[
            pl.BlockSpec((1, N, HID), lambda g: (g, 0, 0)),
            full((N, HID)), full((HID, HID)), full((1, HID)),
            full((HID, HID)), full((HID, HEADS)), full((HID, HEADS)),
            full((1, HID)),
            full((HID, OUT)), full((OUT, HEADS)), full((OUT, HEADS)),
            full((1, OUT)),
        ],
        out_specs=pl.BlockSpec((1, N, OUT), lambda g: (g, 0, 0)),
        out_shape=jax.ShapeDtypeStruct((B * S, N, OUT), jnp.float32),
        scratch_shapes=[pltpu.VMEM((N, N), jnp.float32)],
    )(xp, p['emb'], prjW, p['proj_b'].reshape(1, HID),
      W1, As1, Ad1, p['gat1_b'].reshape(1, HID),
      W2, As2, Ad2, p['gat2_b'].reshape(1, OUT))

    L = B * N
    Tin = h_out.reshape(L, SEQ, OUT).transpose(1, 2, 0).reshape(SEQ * OUT, L)

    eye8 = jnp.eye(SEQ, dtype=jnp.float32)
    bd = lambda w: jnp.kron(eye8, w.T)
    col = lambda v, rep: jnp.tile(v.reshape(-1), rep).reshape(-1, 1)

    Wq = bd(p['Wq'].reshape(16, 16))
    Wk = bd(p['Wk'].reshape(16, 16))
    Wv = bd(p['Wv'].reshape(16, 16))
    Wo = bd(p['Wo'].reshape(16, 16))
    W1f = bd(p['ffn_W1'])
    W2f = bd(p['ffn_W2'])

    out = pl.pallas_call(
        _stage_b_body,
        in_specs=[
            full((SEQ * OUT, L)),
            full((128, 128)), full((128, 1)),
            full((128, 128)), full((128, 1)),
            full((128, 128)), full((128, 1)),
            full((128, 128)), full((128, 1)),
            full((256, 128)), full((256, 1)),
            full((128, 256)), full((128, 1)),
            full((128, 1)), full((128, 1)), full((128, 1)), full((128, 1)),
            full((16, 16)), full((16, 1)), full((16, 1)), full((1, 1)),
        ],
        out_specs=full((1, L)),
        out_shape=jax.ShapeDtypeStruct((1, L), jnp.float32),
        grid=(1,),
    )(Tin,
      Wq, col(p['bq'], SEQ), Wk, col(p['bk'], SEQ), Wv, col(p['bv'], SEQ),
      Wo, col(p['bo'], SEQ),
      W1f, col(p['ffn_b1'], SEQ), W2f, col(p['ffn_b2'], SEQ),
      col(p['ln1_g'], SEQ), col(p['ln1_b'], SEQ),
      col(p['ln2_g'], SEQ), col(p['ln2_b'], SEQ),
      p['reg_W1'].T, p['reg_b1'].reshape(16, 1),
      p['reg_W2'].reshape(16, 1), p['reg_b2'].reshape(1, 1))

    return out.reshape(B, N)
